# baseline (speedup 1.0000x reference)
import jax
import jax.numpy as jnp
from jax.experimental import pallas as pl

_TILE = 3584
_SPLITS = 4


def _make_proj_kernel(splits):
    def _proj_kernel(*refs):
        xs = refs[:splits]
        rms = refs[splits:2 * splits]
        o_ref = refs[2 * splits]
        acc = None
        for x_ref, rm_ref in zip(xs, rms):
            part = jax.lax.dot_general(
                x_ref[0].astype(jnp.bfloat16),
                rm_ref[0].astype(jnp.bfloat16),
                dimension_numbers=(((0,), (0,)), ((), ())),
                preferred_element_type=jnp.float32,
            )
            acc = part if acc is None else acc + part
        o_ref[0] = acc

    return _proj_kernel


def kernel(inputs, random_matrices):
    n, c, h, w = inputs.shape
    hw = h * w
    m = random_matrices.shape[2]
    x = inputs.reshape(n, c, hw)

    tile = _TILE if hw % _TILE == 0 else hw
    splits = _SPLITS if c % _SPLITS == 0 else 1
    cs = c // splits
    grid = (n, hw // tile)

    def x_map(s):
        return lambda b, t: (b, s, t)

    def rm_map(s):
        return lambda b, t: (b, s, 0)

    in_specs = (
        [pl.BlockSpec((1, cs, tile), x_map(s)) for s in range(splits)]
        + [pl.BlockSpec((1, cs, m), rm_map(s)) for s in range(splits)]
    )

    return pl.pallas_call(
        _make_proj_kernel(splits),
        grid=grid,
        in_specs=in_specs,
        out_specs=pl.BlockSpec((1, tile, m), lambda b, t: (b, t, 0)),
        out_shape=jax.ShapeDtypeStruct((n, hw, m), jnp.float32),
    )(*([x] * splits + [random_matrices] * splits))


# speedup vs baseline: 1.0019x; 1.0019x over previous
import jax
import jax.numpy as jnp
from jax.experimental import pallas as pl
from jax.experimental.pallas import tpu as pltpu

_TILE = 3584
_SPLITS = 4


def _make_proj_kernel(splits):
    def _proj_kernel(*refs):
        xs = refs[:splits]
        rms = refs[splits:2 * splits]
        o_ref = refs[2 * splits]
        acc = None
        for x_ref, rm_ref in zip(xs, rms):
            part = jax.lax.dot_general(
                x_ref[0].astype(jnp.bfloat16),
                rm_ref[0].astype(jnp.bfloat16),
                dimension_numbers=(((0,), (0,)), ((), ())),
                preferred_element_type=jnp.float32,
            )
            acc = part if acc is None else acc + part
        o_ref[0] = acc

    return _proj_kernel


def kernel(inputs, random_matrices):
    n, c, h, w = inputs.shape
    hw = h * w
    m = random_matrices.shape[2]
    x = inputs.reshape(n, c, hw)

    tile = _TILE if hw % _TILE == 0 else hw
    splits = _SPLITS if c % _SPLITS == 0 else 1
    cs = c // splits
    grid = (n, hw // tile)

    def x_map(s):
        return lambda b, t: (b, s, t)

    def rm_map(s):
        return lambda b, t: (b, s, 0)

    in_specs = (
        [pl.BlockSpec((1, cs, tile), x_map(s)) for s in range(splits)]
        + [pl.BlockSpec((1, cs, m), rm_map(s)) for s in range(splits)]
    )

    return pl.pallas_call(
        _make_proj_kernel(splits),
        grid=grid,
        in_specs=in_specs,
        out_specs=pl.BlockSpec((1, tile, m), lambda b, t: (b, t, 0)),
        out_shape=jax.ShapeDtypeStruct((n, hw, m), jnp.float32),
        compiler_params=pltpu.CompilerParams(
            dimension_semantics=("parallel", "parallel"),
        ),
    )(*([x] * splits + [random_matrices] * splits))


# speedup vs baseline: 3.0661x; 3.0603x over previous
import jax
import jax.numpy as jnp
from jax.experimental import pallas as pl
from jax.experimental.pallas import tpu as pltpu

_TILE = 3584


def _proj_kernel(x_ref, rm_ref, o_ref):
    o_ref[0] = jax.lax.dot_general(
        x_ref[0].astype(jnp.bfloat16),
        rm_ref[0].astype(jnp.bfloat16),
        dimension_numbers=(((1,), (0,)), ((), ())),
        preferred_element_type=jnp.float32,
    )


def kernel(inputs, random_matrices):
    n, c, h, w = inputs.shape
    hw = h * w
    m = random_matrices.shape[2]
    xt = inputs.reshape(n, c, hw).transpose(0, 2, 1)

    tile = _TILE if hw % _TILE == 0 else hw
    grid = (n, hw // tile)

    return pl.pallas_call(
        _proj_kernel,
        grid=grid,
        in_specs=[
            pl.BlockSpec((1, tile, c), lambda b, t: (b, t, 0)),
            pl.BlockSpec((1, c, m), lambda b, t: (b, 0, 0)),
        ],
        out_specs=pl.BlockSpec((1, tile, m), lambda b, t: (b, t, 0)),
        out_shape=jax.ShapeDtypeStruct((n, hw, m), jnp.float32),
        compiler_params=pltpu.CompilerParams(
            dimension_semantics=("parallel", "parallel"),
        ),
    )(xt, random_matrices)


# speedup vs baseline: 3.1938x; 1.0417x over previous
import jax
import jax.numpy as jnp
from jax.experimental import pallas as pl
from jax.experimental.pallas import tpu as pltpu

_TILE = 7168


def _proj_kernel(x_ref, rm_ref, o_ref):
    o_ref[0] = jax.lax.dot_general(
        x_ref[0].astype(jnp.bfloat16),
        rm_ref[0].astype(jnp.bfloat16),
        dimension_numbers=(((1,), (0,)), ((), ())),
        preferred_element_type=jnp.float32,
    )


def kernel(inputs, random_matrices):
    n, c, h, w = inputs.shape
    hw = h * w
    m = random_matrices.shape[2]
    xt = inputs.reshape(n, c, hw).transpose(0, 2, 1)

    tile = _TILE if hw % _TILE == 0 else hw
    grid = (n, hw // tile)

    return pl.pallas_call(
        _proj_kernel,
        grid=grid,
        in_specs=[
            pl.BlockSpec((1, tile, c), lambda b, t: (b, t, 0)),
            pl.BlockSpec((1, c, m), lambda b, t: (b, 0, 0)),
        ],
        out_specs=pl.BlockSpec((1, tile, m), lambda b, t: (b, t, 0)),
        out_shape=jax.ShapeDtypeStruct((n, hw, m), jnp.float32),
        compiler_params=pltpu.CompilerParams(
            dimension_semantics=("parallel", "parallel"),
        ),
    )(xt, random_matrices)
